# baseline (speedup 1.0000x reference)
import functools

import jax
import jax.numpy as jnp
from jax import lax
from jax.experimental import pallas as pl
from jax.experimental.pallas import tpu as pltpu
from jax.experimental.pallas import tpu_sc as plsc

NUM_EMBEDDINGS = 1000000
FEATURES = 32
BATCH = 4096
LENGTH = 200

NC = 2
NS = 16
NW = NC * NS

FT = 4
FS = 8
BT = 32
BL = 128

SB = 4
BTS = BT // SB
CHUNK = BTS * BL
NBLK = LENGTH * SB
BLK_PER_W = NBLK // NW


def _make_gather():
    mesh = plsc.VectorSubcoreMesh(
        core_axis_name="c", subcore_axis_name="s", num_cores=NC, num_subcores=NS
    )

    @functools.partial(
        pl.kernel,
        out_type=jax.ShapeDtypeStruct((LENGTH, FT, BT, FS, BL), jnp.float32),
        mesh=mesh,
        compiler_params=pltpu.CompilerParams(
            use_tc_tiling_on_sc=False, needs_layout_passes=False
        ),
        scratch_types=[
            pltpu.VMEM((CHUNK,), jnp.int32),
            pltpu.VMEM((CHUNK, FEATURES), jnp.float32),
            pltpu.VMEM((FT, BTS, FS, BL), jnp.float32),
            pltpu.SemaphoreType.DMA,
            pltpu.SemaphoreType.DMA,
            pltpu.SemaphoreType.DMA,
        ],
    )
    def k(table_hbm, idx_hbm, out_hbm, idx_v, rows_v, tbuf_v, isem, gsem, wsem):
        wid = lax.axis_index("s") * NC + lax.axis_index("c")
        lane = jnp.arange(16, dtype=jnp.int32)

        def block_body(i, carry):
            m = wid * BLK_PER_W + i
            l = m // SB
            sb = m % SB
            pltpu.async_copy(
                idx_hbm.at[l, pl.ds(sb * CHUNK, CHUNK)], idx_v, isem
            ).wait()
            pltpu.async_copy(table_hbm.at[idx_v], rows_v, gsem).wait()

            def tr_body(g, carry2):
                rows16 = g * 16 + lane
                btr = g // 8
                blo = (g % 8) * 16
                for f in range(FEATURES):
                    v = plsc.load_gather(
                        rows_v, [rows16, jnp.full((16,), f, dtype=jnp.int32)]
                    )
                    tbuf_v[f // FS, btr, f % FS, pl.ds(blo, 16)] = v
                return carry2

            lax.fori_loop(0, CHUNK // 16, tr_body, 0)

            descs = [
                pltpu.async_copy(
                    tbuf_v.at[ft],
                    out_hbm.at[l, ft, pl.ds(sb * BTS, BTS)],
                    wsem,
                )
                for ft in range(FT)
            ]
            for d in descs:
                d.wait()
            return carry

        lax.fori_loop(0, BLK_PER_W, block_body, 0)

    return k


_gather = _make_gather()


def kernel(inputs, embedding):
    idx = jnp.transpose(inputs)
    out5 = _gather(embedding, idx)
    return out5.transpose(2, 4, 0, 1, 3).reshape(BATCH, LENGTH, FEATURES)
